# Initial kernel scaffold; baseline (speedup 1.0000x reference)
import jax
import jax.numpy as jnp
from jax.experimental import pallas as pl


def kernel(x, edge_index, edge_attr, batch, W1, a_src1, a_dst1, b1, Wm, a_srcm, a_dstm, bm, W2, a_src2, a_dst2, b2, lin1_W, lin2_W):
    raise NotImplementedError("write your pallas kernel here")


# speedup vs baseline: 51.3682x; 51.3682x over previous
import functools

import jax
import jax.numpy as jnp
from jax import lax
from jax.experimental import pallas as pl
from jax.experimental.pallas import tpu as pltpu
from jax.experimental.pallas import tpu_sc as plsc

N = 10000
NP = 10240
E = 160000
EP = 172032
HEADS = 8
HID = 16

NC = 2
NS = 16
NW = NC * NS
C_TILE = EP // NW
NSL = C_TILE // 128
SUB = 128
NSUB = C_TILE // SUB

ROWS16 = NP // NS


@functools.cache
def _get_mesh():
    return plsc.VectorSubcoreMesh(
        core_axis_name="c", subcore_axis_name="s",
        num_cores=NC, num_subcores=NS)


def _matmul_attn_body(x_ref, w_ref, amat_ref, h_ref, asd_ref, g_ref, gscr):
    pi = pl.program_id(0)

    @pl.when(pi == 0)
    def _():
        gscr[...] = jnp.full((1, 8), -1e30, jnp.float32)

    h = jax.lax.dot_general(
        x_ref[...], w_ref[...], (((1,), (0,)), ((), ())),
        preferred_element_type=jnp.float32,
        precision=jax.lax.Precision.HIGHEST)
    asd = jax.lax.dot_general(
        h, amat_ref[...], (((1,), (0,)), ((), ())),
        preferred_element_type=jnp.float32,
        precision=jax.lax.Precision.HIGHEST)
    h_ref[...] = h
    asd_ref[...] = asd
    gscr[...] = jnp.maximum(gscr[...],
                            jnp.max(asd[:, :8], axis=0, keepdims=True))
    g_ref[...] = gscr[...]


def _norm_matmul_attn_body(o0_ref, o1_ref, s0_ref, s1_ref, xmat_ref, b_ref,
                           w_ref, amat_ref, h_ref, asd_ref, g_ref, gscr):
    pi = pl.program_id(0)

    @pl.when(pi == 0)
    def _():
        gscr[...] = jnp.full((1, 8), -1e30, jnp.float32)

    s = s0_ref[...] + s1_ref[...]
    r8 = 1.0 / (s[:, :8] + 1e-16)
    rexp = jax.lax.dot_general(
        r8, xmat_ref[...], (((1,), (0,)), ((), ())),
        preferred_element_type=jnp.float32,
        precision=jax.lax.Precision.HIGHEST)
    x = (o0_ref[...] + o1_ref[...]) * rexp + b_ref[...]
    x = jnp.where(x > 0, x, jnp.exp(x) - 1.0)
    h = jax.lax.dot_general(
        x, w_ref[...], (((1,), (0,)), ((), ())),
        preferred_element_type=jnp.float32,
        precision=jax.lax.Precision.HIGHEST)
    asd = jax.lax.dot_general(
        h, amat_ref[...], (((1,), (0,)), ((), ())),
        preferred_element_type=jnp.float32,
        precision=jax.lax.Precision.HIGHEST)
    h_ref[...] = h
    asd_ref[...] = asd
    gscr[...] = jnp.maximum(gscr[...],
                            jnp.max(asd[:, :8], axis=0, keepdims=True))
    g_ref[...] = gscr[...]


def _tc_layer_pre(x, w, amat, fused_inputs=None):
    fout = w.shape[1]
    nblk = NP // 256
    out_shapes = (
        jax.ShapeDtypeStruct((NP, fout), jnp.float32),
        jax.ShapeDtypeStruct((NP, 16), jnp.float32),
        jax.ShapeDtypeStruct((1, 8), jnp.float32),
    )
    row_spec = lambda width: pl.BlockSpec((256, width), lambda i: (i, 0))
    full = lambda a: pl.BlockSpec(a.shape, lambda i: (0,) * a.ndim)
    out_specs = (row_spec(fout), row_spec(16),
                 pl.BlockSpec((1, 8), lambda i: (0, 0)))
    scratch = [pltpu.VMEM((1, 8), jnp.float32)]
    if fused_inputs is None:
        return pl.pallas_call(
            _matmul_attn_body, grid=(nblk,),
            in_specs=[row_spec(x.shape[1]), full(w), full(amat)],
            out_specs=out_specs, out_shape=out_shapes,
            scratch_shapes=scratch,
        )(x, w, amat)
    o2, s2, xmat, b = fused_inputs
    fin = o2.shape[2]
    return pl.pallas_call(
        _norm_matmul_attn_body, grid=(nblk,),
        in_specs=[row_spec(fin), row_spec(fin),
                  row_spec(16), row_spec(16),
                  full(xmat),
                  pl.BlockSpec((1, b.shape[1]), lambda i: (0, 0)),
                  full(w), full(amat)],
        out_specs=out_specs, out_shape=out_shapes,
        scratch_shapes=scratch,
    )(o2[0], o2[1], s2[0], s2[1], xmat, b, w, amat)


def _head_body(o0_ref, o1_ref, s0_ref, s1_ref, xmat_ref, b_ref,
               w1_ref, w2_ref, out_ref):
    s = s0_ref[...] + s1_ref[...]
    r8 = 1.0 / (s[:, :8] + 1e-16)
    rexp = jax.lax.dot_general(
        r8, xmat_ref[...], (((1,), (0,)), ((), ())),
        preferred_element_type=jnp.float32,
        precision=jax.lax.Precision.HIGHEST)
    x = (o0_ref[...] + o1_ref[...]) * rexp + b_ref[...]
    x = jax.lax.dot_general(
        x, w1_ref[...], (((1,), (0,)), ((), ())),
        preferred_element_type=jnp.float32,
        precision=jax.lax.Precision.HIGHEST)
    x = jnp.where(x > 0, x, jnp.exp(x) - 1.0)
    x = jax.lax.dot_general(
        x, w2_ref[...], (((1,), (0,)), ((), ())),
        preferred_element_type=jnp.float32,
        precision=jax.lax.Precision.HIGHEST)
    out_ref[...] = jax.nn.sigmoid(x)


def _tc_head(o2, s2, xmat, b2, lin1_w, lin2_w8):
    nblk = NP // 256
    return pl.pallas_call(
        _head_body, grid=(nblk,),
        in_specs=[pl.BlockSpec((256, 64), lambda i: (i, 0)),
                  pl.BlockSpec((256, 64), lambda i: (i, 0)),
                  pl.BlockSpec((256, 16), lambda i: (i, 0)),
                  pl.BlockSpec((256, 16), lambda i: (i, 0)),
                  pl.BlockSpec((8, 64), lambda i: (0, 0)),
                  pl.BlockSpec((1, 64), lambda i: (0, 0)),
                  pl.BlockSpec((64, 8), lambda i: (0, 0)),
                  pl.BlockSpec((8, 8), lambda i: (0, 0))],
        out_specs=pl.BlockSpec((256, 8), lambda i: (i, 0)),
        out_shape=jax.ShapeDtypeStruct((NP, 8), jnp.float32),
    )(o2[0], o2[1], s2[0], s2[1], xmat, b2, lin1_w, lin2_w8)


def _sc_layer(ngrp, src_h, dst_h, asd_h, g_h, hf_h, z16_h, zf_h,
              s_out, o_out, si, di, bs, bd, hb, gv, ssh, osh, sem):
    fw = ngrp * 16
    oc = fw // HEADS
    cid = lax.axis_index("c")
    sid = lax.axis_index("s")
    wid = sid * NC + cid
    pltpu.sync_copy(z16_h.at[pl.ds(sid * ROWS16, ROWS16)],
                    ssh.at[pl.ds(sid * ROWS16, ROWS16)])
    pltpu.sync_copy(zf_h.at[pl.ds(sid * ROWS16, ROWS16)],
                    osh.at[pl.ds(sid * ROWS16, ROWS16)])
    pltpu.sync_copy(src_h.at[wid], si)
    pltpu.sync_copy(dst_h.at[wid], di)
    pltpu.sync_copy(g_h, gv)
    plsc.subcore_barrier()

    gvec = gv[...]
    iot = lax.iota(jnp.int32, 16)
    rot = (iot + 8) & 15
    half = iot < 8
    gidx = [g * (16 // oc) + iot // oc for g in range(ngrp)]

    def sub(t, _):
        d1 = pltpu.async_copy(asd_h.at[si.at[t]], bs, sem)
        d2 = pltpu.async_copy(asd_h.at[di.at[t]], bd, sem)
        d3 = pltpu.async_copy(hf_h.at[si.at[t]], hb, sem)
        d1.wait()
        d2.wait()
        d3.wait()

        def edge(i, _):
            vs = bs[i, :]
            vd = bd[i, :]
            vsr = vs.at[rot].get(mode="promise_in_bounds")
            vdr = vd.at[rot].get(mode="promise_in_bounds")
            as16 = jnp.where(half, vs, vsr)
            ad16 = jnp.where(half, vdr, vd)
            x = as16 + ad16
            e = jnp.where(x >= 0, x, 0.2 * x)
            m = ad16 + gvec
            m = jnp.where(m >= 0, m, 0.2 * m)
            p = jnp.exp(e - m)
            bs[i, :] = p
            for g in range(ngrp):
                a = p.at[gidx[g]].get(mode="promise_in_bounds")
                hb[i, pl.ds(g * 16, 16)] = hb[i, pl.ds(g * 16, 16)] * a
            return 0

        lax.fori_loop(0, SUB, edge, 0)
        pltpu.sync_copy(bs, ssh.at[di.at[t]], add=True)
        pltpu.sync_copy(hb, osh.at[di.at[t]], add=True)
        return 0

    lax.fori_loop(0, NSUB, sub, 0)
    plsc.subcore_barrier()
    pltpu.sync_copy(ssh.at[pl.ds(sid * ROWS16, ROWS16)],
                    s_out.at[cid, pl.ds(sid * ROWS16, ROWS16)])
    pltpu.sync_copy(osh.at[pl.ds(sid * ROWS16, ROWS16)],
                    o_out.at[cid, pl.ds(sid * ROWS16, ROWS16)])


def _sc_gat_layer(srcv, dstv, asd, g, hf, z16, zf):
    fw = hf.shape[1]
    ngrp = fw // 16
    g16 = jnp.tile(g.reshape(8), 2)
    f = functools.partial(
        pl.kernel, mesh=_get_mesh(),
        compiler_params=pltpu.CompilerParams(
            use_tc_tiling_on_sc=False, needs_layout_passes=False),
        out_type=(jax.ShapeDtypeStruct((NC, NP, 16), jnp.float32),
                  jax.ShapeDtypeStruct((NC, NP, fw), jnp.float32)),
        scratch_types=[
            pltpu.VMEM((NSL, 128), jnp.int32),
            pltpu.VMEM((NSL, 128), jnp.int32),
            pltpu.VMEM((SUB, 16), jnp.float32),
            pltpu.VMEM((SUB, 16), jnp.float32),
            pltpu.VMEM((SUB, fw), jnp.float32),
            pltpu.VMEM((16,), jnp.float32),
            pltpu.VMEM_SHARED((NP, 16), jnp.float32),
            pltpu.VMEM_SHARED((NP, fw), jnp.float32),
            pltpu.SemaphoreType.DMA,
        ])(functools.partial(_sc_layer, ngrp))
    return f(srcv, dstv, asd, g16, hf, z16, zf)


def _head_mat(a):
    heads, oc = a.shape
    eye = jnp.eye(heads, dtype=a.dtype)
    return (a[:, :, None] * eye[:, None, :]).reshape(heads * oc, heads)


def kernel(x, edge_index, edge_attr, batch, W1, a_src1, a_dst1, b1,
           Wm, a_srcm, a_dstm, bm, W2, a_src2, a_dst2, b2, lin1_W, lin2_W):
    del edge_attr, batch
    f32 = jnp.float32

    loops = jnp.arange(N, dtype=jnp.int32)
    src = jnp.concatenate([edge_index[0], loops])
    dst = jnp.concatenate([edge_index[1], loops])
    pad_e = EP - (E + N)
    pad_idx = jnp.full((pad_e,), NP - 1, jnp.int32)
    srcv = jnp.concatenate([src, pad_idx]).reshape(NW, NSL, 128)
    dstv = jnp.concatenate([dst, pad_idx]).reshape(NW, NSL, 128)

    xp = jnp.pad(x, ((0, NP - N), (0, 2))).astype(f32)
    W1p = jnp.pad(W1, ((0, 2), (0, 0)))
    amat1 = jnp.concatenate([_head_mat(a_src1), _head_mat(a_dst1)], axis=1)
    amatm = jnp.concatenate([_head_mat(a_srcm), _head_mat(a_dstm)], axis=1)
    amat2 = jnp.concatenate([_head_mat(a_src2), _head_mat(a_dst2)], axis=1)
    lin2_W8 = jnp.pad(lin2_W, ((0, 0), (0, 7)))
    xmat128 = jnp.kron(jnp.eye(8, dtype=f32), jnp.ones((1, 16), f32))
    xmat64 = jnp.kron(jnp.eye(8, dtype=f32), jnp.ones((1, 8), f32))

    z16 = jnp.zeros((NP, 16), f32)
    z128 = jnp.zeros((NP, 128), f32)
    z64 = jnp.zeros((NP, 64), f32)

    h1, asd1, g1 = _tc_layer_pre(xp, W1p, amat1)
    s1, o1 = _sc_gat_layer(srcv, dstv, asd1, g1, h1, z16, z128)
    h2, asd2, g2 = _tc_layer_pre(
        None, Wm, amatm,
        fused_inputs=(o1, s1, xmat128, b1.reshape(1, 128)))
    s2, o2 = _sc_gat_layer(srcv, dstv, asd2, g2, h2, z16, z128)
    h3, asd3, g3 = _tc_layer_pre(
        None, W2, amat2,
        fused_inputs=(o2, s2, xmat128, bm.reshape(1, 128)))
    s3, o3 = _sc_gat_layer(srcv, dstv, asd3, g3, h3, z16, z64)
    out = _tc_head(o3, s3, xmat64, b2.reshape(1, 64), lin1_W, lin2_W8)
    return out[:N, :1]


# speedup vs baseline: 63.9427x; 1.2448x over previous
import functools

import jax
import jax.numpy as jnp
from jax import lax
from jax.experimental import pallas as pl
from jax.experimental.pallas import tpu as pltpu
from jax.experimental.pallas import tpu_sc as plsc

N = 10000
NP = 10240
E = 160000
EP = 172032
HEADS = 8
HID = 16

NC = 2
NS = 16
NW = NC * NS
C_TILE = EP // NW
NSL = C_TILE // 128
SUB = 64
NSUB = C_TILE // SUB

ROWS16 = NP // NS


@functools.cache
def _get_mesh():
    return plsc.VectorSubcoreMesh(
        core_axis_name="c", subcore_axis_name="s",
        num_cores=NC, num_subcores=NS)


def _matmul_attn_body(x_ref, w_ref, amat_ref, h_ref, asd_ref, g_ref, gscr):
    pi = pl.program_id(0)

    @pl.when(pi == 0)
    def _():
        gscr[...] = jnp.full((1, 8), -1e30, jnp.float32)

    h = jax.lax.dot_general(
        x_ref[...], w_ref[...], (((1,), (0,)), ((), ())),
        preferred_element_type=jnp.float32,
        precision=jax.lax.Precision.HIGHEST)
    asd = jax.lax.dot_general(
        h, amat_ref[...], (((1,), (0,)), ((), ())),
        preferred_element_type=jnp.float32,
        precision=jax.lax.Precision.HIGHEST)
    h_ref[...] = h
    asd_ref[...] = asd
    gscr[...] = jnp.maximum(gscr[...],
                            jnp.max(asd[:, :8], axis=0, keepdims=True))
    g_ref[...] = gscr[...]


def _norm_matmul_attn_body(o0_ref, o1_ref, s0_ref, s1_ref, xmat_ref, b_ref,
                           w_ref, amat_ref, h_ref, asd_ref, g_ref, gscr):
    pi = pl.program_id(0)

    @pl.when(pi == 0)
    def _():
        gscr[...] = jnp.full((1, 8), -1e30, jnp.float32)

    s = s0_ref[...] + s1_ref[...]
    r8 = 1.0 / (s[:, :8] + 1e-16)
    rexp = jax.lax.dot_general(
        r8, xmat_ref[...], (((1,), (0,)), ((), ())),
        preferred_element_type=jnp.float32,
        precision=jax.lax.Precision.HIGHEST)
    x = (o0_ref[...] + o1_ref[...]) * rexp + b_ref[...]
    x = jnp.where(x > 0, x, jnp.exp(x) - 1.0)
    h = jax.lax.dot_general(
        x, w_ref[...], (((1,), (0,)), ((), ())),
        preferred_element_type=jnp.float32,
        precision=jax.lax.Precision.HIGHEST)
    asd = jax.lax.dot_general(
        h, amat_ref[...], (((1,), (0,)), ((), ())),
        preferred_element_type=jnp.float32,
        precision=jax.lax.Precision.HIGHEST)
    h_ref[...] = h
    asd_ref[...] = asd
    gscr[...] = jnp.maximum(gscr[...],
                            jnp.max(asd[:, :8], axis=0, keepdims=True))
    g_ref[...] = gscr[...]


def _tc_layer_pre(x, w, amat, fused_inputs=None):
    fout = w.shape[1]
    nblk = NP // 256
    out_shapes = (
        jax.ShapeDtypeStruct((NP, fout), jnp.float32),
        jax.ShapeDtypeStruct((NP, 16), jnp.float32),
        jax.ShapeDtypeStruct((1, 8), jnp.float32),
    )
    row_spec = lambda width: pl.BlockSpec((256, width), lambda i: (i, 0))
    full = lambda a: pl.BlockSpec(a.shape, lambda i: (0,) * a.ndim)
    out_specs = (row_spec(fout), row_spec(16),
                 pl.BlockSpec((1, 8), lambda i: (0, 0)))
    scratch = [pltpu.VMEM((1, 8), jnp.float32)]
    if fused_inputs is None:
        return pl.pallas_call(
            _matmul_attn_body, grid=(nblk,),
            in_specs=[row_spec(x.shape[1]), full(w), full(amat)],
            out_specs=out_specs, out_shape=out_shapes,
            scratch_shapes=scratch,
        )(x, w, amat)
    o2, s2, xmat, b = fused_inputs
    fin = o2.shape[2]
    return pl.pallas_call(
        _norm_matmul_attn_body, grid=(nblk,),
        in_specs=[row_spec(fin), row_spec(fin),
                  row_spec(16), row_spec(16),
                  full(xmat),
                  pl.BlockSpec((1, b.shape[1]), lambda i: (0, 0)),
                  full(w), full(amat)],
        out_specs=out_specs, out_shape=out_shapes,
        scratch_shapes=scratch,
    )(o2[0], o2[1], s2[0], s2[1], xmat, b, w, amat)


def _head_body(o0_ref, o1_ref, s0_ref, s1_ref, xmat_ref, b_ref,
               w1_ref, w2_ref, out_ref):
    s = s0_ref[...] + s1_ref[...]
    r8 = 1.0 / (s[:, :8] + 1e-16)
    rexp = jax.lax.dot_general(
        r8, xmat_ref[...], (((1,), (0,)), ((), ())),
        preferred_element_type=jnp.float32,
        precision=jax.lax.Precision.HIGHEST)
    x = (o0_ref[...] + o1_ref[...]) * rexp + b_ref[...]
    x = jax.lax.dot_general(
        x, w1_ref[...], (((1,), (0,)), ((), ())),
        preferred_element_type=jnp.float32,
        precision=jax.lax.Precision.HIGHEST)
    x = jnp.where(x > 0, x, jnp.exp(x) - 1.0)
    x = jax.lax.dot_general(
        x, w2_ref[...], (((1,), (0,)), ((), ())),
        preferred_element_type=jnp.float32,
        precision=jax.lax.Precision.HIGHEST)
    out_ref[...] = jax.nn.sigmoid(x)


def _tc_head(o2, s2, xmat, b2, lin1_w, lin2_w8):
    nblk = NP // 256
    return pl.pallas_call(
        _head_body, grid=(nblk,),
        in_specs=[pl.BlockSpec((256, 64), lambda i: (i, 0)),
                  pl.BlockSpec((256, 64), lambda i: (i, 0)),
                  pl.BlockSpec((256, 16), lambda i: (i, 0)),
                  pl.BlockSpec((256, 16), lambda i: (i, 0)),
                  pl.BlockSpec((8, 64), lambda i: (0, 0)),
                  pl.BlockSpec((1, 64), lambda i: (0, 0)),
                  pl.BlockSpec((64, 8), lambda i: (0, 0)),
                  pl.BlockSpec((8, 8), lambda i: (0, 0))],
        out_specs=pl.BlockSpec((256, 8), lambda i: (i, 0)),
        out_shape=jax.ShapeDtypeStruct((NP, 8), jnp.float32),
    )(o2[0], o2[1], s2[0], s2[1], xmat, b2, lin1_w, lin2_w8)


def _sc_layer(ngrp, src_h, dst_h, asd_h, g_h, hf_h, z16_h, zf_h,
              s_out, o_out, si, di, bs, bd, hb, gv, ssh, osh, semg, sems):
    fw = ngrp * 16
    oc = fw // HEADS
    cid = lax.axis_index("c")
    sid = lax.axis_index("s")
    wid = sid * NC + cid
    pltpu.sync_copy(z16_h.at[pl.ds(sid * ROWS16, ROWS16)],
                    ssh.at[pl.ds(sid * ROWS16, ROWS16)])
    pltpu.sync_copy(zf_h.at[pl.ds(sid * ROWS16, ROWS16)],
                    osh.at[pl.ds(sid * ROWS16, ROWS16)])
    pltpu.sync_copy(src_h.at[wid], si)
    pltpu.sync_copy(dst_h.at[wid], di)
    pltpu.sync_copy(g_h, gv)
    plsc.subcore_barrier()

    gvec = gv[...]
    iot = lax.iota(jnp.int32, 16)
    rot = (iot + 8) & 15
    half = iot < 8
    gidx = [g * (16 // oc) + iot // oc for g in range(ngrp)]

    def fire_gathers(t, par):
        pltpu.async_copy(asd_h.at[si.at[t]], bs.at[par], semg)
        pltpu.async_copy(asd_h.at[di.at[t]], bd.at[par], semg)
        pltpu.async_copy(hf_h.at[si.at[t]], hb.at[par], semg)

    def wait_gathers(t, par):
        pltpu.make_async_copy(asd_h.at[si.at[t]], bs.at[par], semg).wait()
        pltpu.make_async_copy(asd_h.at[di.at[t]], bd.at[par], semg).wait()
        pltpu.make_async_copy(hf_h.at[si.at[t]], hb.at[par], semg).wait()

    def fire_scatters(t, par):
        pltpu.async_copy(bs.at[par], ssh.at[di.at[t]], sems, add=True)
        pltpu.async_copy(hb.at[par], osh.at[di.at[t]], sems, add=True)

    def wait_scatters(t, par):
        pltpu.make_async_copy(bs.at[par], ssh.at[di.at[t]], sems).wait()
        pltpu.make_async_copy(hb.at[par], osh.at[di.at[t]], sems).wait()

    fire_gathers(0, 0)

    def sub(t, _):
        par = t & 1
        wait_gathers(t, par)

        @pl.when(t > 0)
        def _():
            wait_scatters(t - 1, 1 - par)

        @pl.when(t < NSUB - 1)
        def _():
            fire_gathers(t + 1, 1 - par)

        def edge(i, _):
            vs = bs[par, i, :]
            vd = bd[par, i, :]
            vsr = vs.at[rot].get(mode="promise_in_bounds")
            vdr = vd.at[rot].get(mode="promise_in_bounds")
            as16 = jnp.where(half, vs, vsr)
            ad16 = jnp.where(half, vdr, vd)
            x = as16 + ad16
            e = jnp.where(x >= 0, x, 0.2 * x)
            m = ad16 + gvec
            m = jnp.where(m >= 0, m, 0.2 * m)
            p = jnp.exp(e - m)
            bs[par, i, :] = p
            for g in range(ngrp):
                a = p.at[gidx[g]].get(mode="promise_in_bounds")
                hb[par, i, pl.ds(g * 16, 16)] = (
                    hb[par, i, pl.ds(g * 16, 16)] * a)
            return 0

        lax.fori_loop(0, SUB, edge, 0)
        fire_scatters(t, par)
        return 0

    lax.fori_loop(0, NSUB, sub, 0)
    wait_scatters(NSUB - 1, (NSUB - 1) & 1)
    plsc.subcore_barrier()
    pltpu.sync_copy(ssh.at[pl.ds(sid * ROWS16, ROWS16)],
                    s_out.at[cid, pl.ds(sid * ROWS16, ROWS16)])
    pltpu.sync_copy(osh.at[pl.ds(sid * ROWS16, ROWS16)],
                    o_out.at[cid, pl.ds(sid * ROWS16, ROWS16)])


def _sc_gat_layer(srcv, dstv, asd, g, hf, z16, zf):
    fw = hf.shape[1]
    ngrp = fw // 16
    g16 = jnp.tile(g.reshape(8), 2)
    f = functools.partial(
        pl.kernel, mesh=_get_mesh(),
        compiler_params=pltpu.CompilerParams(
            use_tc_tiling_on_sc=False, needs_layout_passes=False),
        out_type=(jax.ShapeDtypeStruct((NC, NP, 16), jnp.float32),
                  jax.ShapeDtypeStruct((NC, NP, fw), jnp.float32)),
        scratch_types=[
            pltpu.VMEM((NSUB, SUB), jnp.int32),
            pltpu.VMEM((NSUB, SUB), jnp.int32),
            pltpu.VMEM((2, SUB, 16), jnp.float32),
            pltpu.VMEM((2, SUB, 16), jnp.float32),
            pltpu.VMEM((2, SUB, fw), jnp.float32),
            pltpu.VMEM((16,), jnp.float32),
            pltpu.VMEM_SHARED((NP, 16), jnp.float32),
            pltpu.VMEM_SHARED((NP, fw), jnp.float32),
            pltpu.SemaphoreType.DMA,
            pltpu.SemaphoreType.DMA,
        ])(functools.partial(_sc_layer, ngrp))
    return f(srcv, dstv, asd, g16, hf, z16, zf)


def _head_mat(a):
    heads, oc = a.shape
    eye = jnp.eye(heads, dtype=a.dtype)
    return (a[:, :, None] * eye[:, None, :]).reshape(heads * oc, heads)


def kernel(x, edge_index, edge_attr, batch, W1, a_src1, a_dst1, b1,
           Wm, a_srcm, a_dstm, bm, W2, a_src2, a_dst2, b2, lin1_W, lin2_W):
    del edge_attr, batch
    f32 = jnp.float32

    loops = jnp.arange(N, dtype=jnp.int32)
    src = jnp.concatenate([edge_index[0], loops])
    dst = jnp.concatenate([edge_index[1], loops])
    pad_e = EP - (E + N)
    pad_idx = jnp.full((pad_e,), NP - 1, jnp.int32)
    srcv = jnp.concatenate([src, pad_idx]).reshape(NW, NSUB, SUB)
    dstv = jnp.concatenate([dst, pad_idx]).reshape(NW, NSUB, SUB)

    xp = jnp.pad(x, ((0, NP - N), (0, 2))).astype(f32)
    W1p = jnp.pad(W1, ((0, 2), (0, 0)))
    amat1 = jnp.concatenate([_head_mat(a_src1), _head_mat(a_dst1)], axis=1)
    amatm = jnp.concatenate([_head_mat(a_srcm), _head_mat(a_dstm)], axis=1)
    amat2 = jnp.concatenate([_head_mat(a_src2), _head_mat(a_dst2)], axis=1)
    lin2_W8 = jnp.pad(lin2_W, ((0, 0), (0, 7)))
    xmat128 = jnp.kron(jnp.eye(8, dtype=f32), jnp.ones((1, 16), f32))
    xmat64 = jnp.kron(jnp.eye(8, dtype=f32), jnp.ones((1, 8), f32))

    z16 = jnp.zeros((NP, 16), f32)
    z128 = jnp.zeros((NP, 128), f32)
    z64 = jnp.zeros((NP, 64), f32)

    h1, asd1, g1 = _tc_layer_pre(xp, W1p, amat1)
    s1, o1 = _sc_gat_layer(srcv, dstv, asd1, g1, h1, z16, z128)
    h2, asd2, g2 = _tc_layer_pre(
        None, Wm, amatm,
        fused_inputs=(o1, s1, xmat128, b1.reshape(1, 128)))
    s2, o2 = _sc_gat_layer(srcv, dstv, asd2, g2, h2, z16, z128)
    h3, asd3, g3 = _tc_layer_pre(
        None, W2, amat2,
        fused_inputs=(o2, s2, xmat128, bm.reshape(1, 128)))
    s3, o3 = _sc_gat_layer(srcv, dstv, asd3, g3, h3, z16, z64)
    out = _tc_head(o3, s3, xmat64, b2.reshape(1, 64), lin1_W, lin2_W8)
    return out[:N, :1]


# speedup vs baseline: 100.6818x; 1.5746x over previous
import functools

import jax
import jax.numpy as jnp
from jax import lax
from jax.experimental import pallas as pl
from jax.experimental.pallas import tpu as pltpu
from jax.experimental.pallas import tpu_sc as plsc

N = 10000
NP = 10240
E = 160000
EP = 172032
HEADS = 8
HID = 16

NC = 2
NS = 16
NW = NC * NS
C_TILE = EP // NW
NSL = C_TILE // 128
SUB = 64
NSUB = C_TILE // SUB

ROWS16 = NP // NS


@functools.cache
def _get_mesh():
    return plsc.VectorSubcoreMesh(
        core_axis_name="c", subcore_axis_name="s",
        num_cores=NC, num_subcores=NS)


def _matmul_attn_body(x_ref, w_ref, amat_ref, h_ref, asd_ref, g_ref, gscr):
    pi = pl.program_id(0)

    @pl.when(pi == 0)
    def _():
        gscr[...] = jnp.full((1, 8), -1e30, jnp.float32)

    h = jax.lax.dot_general(
        x_ref[...], w_ref[...], (((1,), (0,)), ((), ())),
        preferred_element_type=jnp.float32)
    asd = jax.lax.dot_general(
        h, amat_ref[...], (((1,), (0,)), ((), ())),
        preferred_element_type=jnp.float32)
    h_ref[...] = h
    asd_ref[...] = asd
    gscr[...] = jnp.maximum(gscr[...],
                            jnp.max(asd[:, :8], axis=0, keepdims=True))
    g_ref[...] = gscr[...]


def _norm_matmul_attn_body(o0_ref, o1_ref, s0_ref, s1_ref, xmat_ref, b_ref,
                           w_ref, amat_ref, h_ref, asd_ref, g_ref, gscr):
    pi = pl.program_id(0)

    @pl.when(pi == 0)
    def _():
        gscr[...] = jnp.full((1, 8), -1e30, jnp.float32)

    s = s0_ref[...] + s1_ref[...]
    r8 = 1.0 / (s[:, :8] + 1e-16)
    rexp = jax.lax.dot_general(
        r8, xmat_ref[...], (((1,), (0,)), ((), ())),
        preferred_element_type=jnp.float32)
    x = (o0_ref[...] + o1_ref[...]) * rexp + b_ref[...]
    x = jnp.where(x > 0, x, jnp.exp(x) - 1.0)
    h = jax.lax.dot_general(
        x, w_ref[...], (((1,), (0,)), ((), ())),
        preferred_element_type=jnp.float32)
    asd = jax.lax.dot_general(
        h, amat_ref[...], (((1,), (0,)), ((), ())),
        preferred_element_type=jnp.float32)
    h_ref[...] = h
    asd_ref[...] = asd
    gscr[...] = jnp.maximum(gscr[...],
                            jnp.max(asd[:, :8], axis=0, keepdims=True))
    g_ref[...] = gscr[...]


def _tc_layer_pre(x, w, amat, fused_inputs=None):
    fout = w.shape[1]
    nblk = NP // 256
    out_shapes = (
        jax.ShapeDtypeStruct((NP, fout), jnp.float32),
        jax.ShapeDtypeStruct((NP, 16), jnp.float32),
        jax.ShapeDtypeStruct((1, 8), jnp.float32),
    )
    row_spec = lambda width: pl.BlockSpec((256, width), lambda i: (i, 0))
    full = lambda a: pl.BlockSpec(a.shape, lambda i: (0,) * a.ndim)
    out_specs = (row_spec(fout), row_spec(16),
                 pl.BlockSpec((1, 8), lambda i: (0, 0)))
    scratch = [pltpu.VMEM((1, 8), jnp.float32)]
    if fused_inputs is None:
        return pl.pallas_call(
            _matmul_attn_body, grid=(nblk,),
            in_specs=[row_spec(x.shape[1]), full(w), full(amat)],
            out_specs=out_specs, out_shape=out_shapes,
            scratch_shapes=scratch,
        )(x, w, amat)
    o2, s2, xmat, b = fused_inputs
    fin = o2.shape[2]
    return pl.pallas_call(
        _norm_matmul_attn_body, grid=(nblk,),
        in_specs=[row_spec(fin), row_spec(fin),
                  row_spec(16), row_spec(16),
                  full(xmat),
                  pl.BlockSpec((1, b.shape[1]), lambda i: (0, 0)),
                  full(w), full(amat)],
        out_specs=out_specs, out_shape=out_shapes,
        scratch_shapes=scratch,
    )(o2[0], o2[1], s2[0], s2[1], xmat, b, w, amat)


def _head_body(o0_ref, o1_ref, s0_ref, s1_ref, xmat_ref, b_ref,
               w1_ref, w2_ref, out_ref):
    s = s0_ref[...] + s1_ref[...]
    r8 = 1.0 / (s[:, :8] + 1e-16)
    rexp = jax.lax.dot_general(
        r8, xmat_ref[...], (((1,), (0,)), ((), ())),
        preferred_element_type=jnp.float32)
    x = (o0_ref[...] + o1_ref[...]) * rexp + b_ref[...]
    x = jax.lax.dot_general(
        x, w1_ref[...], (((1,), (0,)), ((), ())),
        preferred_element_type=jnp.float32)
    x = jnp.where(x > 0, x, jnp.exp(x) - 1.0)
    x = jax.lax.dot_general(
        x, w2_ref[...], (((1,), (0,)), ((), ())),
        preferred_element_type=jnp.float32)
    out_ref[...] = jax.nn.sigmoid(x)


def _tc_head(o2, s2, xmat, b2, lin1_w, lin2_w8):
    nblk = NP // 256
    return pl.pallas_call(
        _head_body, grid=(nblk,),
        in_specs=[pl.BlockSpec((256, 64), lambda i: (i, 0)),
                  pl.BlockSpec((256, 64), lambda i: (i, 0)),
                  pl.BlockSpec((256, 16), lambda i: (i, 0)),
                  pl.BlockSpec((256, 16), lambda i: (i, 0)),
                  pl.BlockSpec((8, 64), lambda i: (0, 0)),
                  pl.BlockSpec((1, 64), lambda i: (0, 0)),
                  pl.BlockSpec((64, 8), lambda i: (0, 0)),
                  pl.BlockSpec((8, 8), lambda i: (0, 0))],
        out_specs=pl.BlockSpec((256, 8), lambda i: (i, 0)),
        out_shape=jax.ShapeDtypeStruct((NP, 8), jnp.float32),
    )(o2[0], o2[1], s2[0], s2[1], xmat, b2, lin1_w, lin2_w8)


def _sc_layer(ngrp, src_h, dst_h, asd_h, g_h, hf_h, z16_h, zf_h,
              s_out, o_out, si, di, bs, bd, hb, gv, ssh, osh, semg, sems):
    fw = ngrp * 16
    oc = fw // HEADS
    cid = lax.axis_index("c")
    sid = lax.axis_index("s")
    wid = sid * NC + cid
    pltpu.sync_copy(z16_h.at[pl.ds(sid * ROWS16, ROWS16)],
                    ssh.at[pl.ds(sid * ROWS16, ROWS16)])
    pltpu.sync_copy(zf_h.at[pl.ds(sid * ROWS16, ROWS16)],
                    osh.at[pl.ds(sid * ROWS16, ROWS16)])
    pltpu.sync_copy(src_h.at[wid], si)
    pltpu.sync_copy(dst_h.at[wid], di)
    pltpu.sync_copy(g_h, gv)
    plsc.subcore_barrier()

    gvec = gv[...]
    iot = lax.iota(jnp.int32, 16)
    rot = (iot + 8) & 15
    half = iot < 8
    gidx = [g * (16 // oc) + iot // oc for g in range(ngrp)]

    def fire_gathers(t, par):
        pltpu.async_copy(asd_h.at[si.at[t]], bs.at[par], semg)
        pltpu.async_copy(asd_h.at[di.at[t]], bd.at[par], semg)
        pltpu.async_copy(hf_h.at[si.at[t]], hb.at[par], semg)

    def wait_gathers(t, par):
        pltpu.make_async_copy(asd_h.at[si.at[t]], bs.at[par], semg).wait()
        pltpu.make_async_copy(asd_h.at[di.at[t]], bd.at[par], semg).wait()
        pltpu.make_async_copy(hf_h.at[si.at[t]], hb.at[par], semg).wait()

    def fire_scatters(t, par):
        pltpu.async_copy(bs.at[par], ssh.at[di.at[t]], sems, add=True)
        pltpu.async_copy(hb.at[par], osh.at[di.at[t]], sems, add=True)

    def wait_scatters(t, par):
        pltpu.make_async_copy(bs.at[par], ssh.at[di.at[t]], sems).wait()
        pltpu.make_async_copy(hb.at[par], osh.at[di.at[t]], sems).wait()

    fire_gathers(0, 0)

    def sub(t, _):
        par = t & 1
        wait_gathers(t, par)

        @pl.when(t > 0)
        def _():
            wait_scatters(t - 1, 1 - par)

        @pl.when(t < NSUB - 1)
        def _():
            fire_gathers(t + 1, 1 - par)

        @plsc.parallel_loop(0, SUB, unroll=16)
        def edge(i):
            vs = bs[par, i, :]
            vd = bd[par, i, :]
            vsr = vs.at[rot].get(mode="promise_in_bounds")
            vdr = vd.at[rot].get(mode="promise_in_bounds")
            as16 = jnp.where(half, vs, vsr)
            ad16 = jnp.where(half, vdr, vd)
            x = as16 + ad16
            e = jnp.where(x >= 0, x, 0.2 * x)
            m = ad16 + gvec
            m = jnp.where(m >= 0, m, 0.2 * m)
            p = jnp.exp(e - m)
            bs[par, i, :] = p
            for g in range(ngrp):
                a = p.at[gidx[g]].get(mode="promise_in_bounds")
                hb[par, i, pl.ds(g * 16, 16)] = (
                    hb[par, i, pl.ds(g * 16, 16)] * a)

        fire_scatters(t, par)
        return 0

    lax.fori_loop(0, NSUB, sub, 0)
    wait_scatters(NSUB - 1, (NSUB - 1) & 1)
    plsc.subcore_barrier()
    pltpu.sync_copy(ssh.at[pl.ds(sid * ROWS16, ROWS16)],
                    s_out.at[cid, pl.ds(sid * ROWS16, ROWS16)])
    pltpu.sync_copy(osh.at[pl.ds(sid * ROWS16, ROWS16)],
                    o_out.at[cid, pl.ds(sid * ROWS16, ROWS16)])


def _sc_gat_layer(srcv, dstv, asd, g, hf, z16, zf):
    fw = hf.shape[1]
    ngrp = fw // 16
    g16 = jnp.tile(g.reshape(8), 2)
    f = functools.partial(
        pl.kernel, mesh=_get_mesh(),
        compiler_params=pltpu.CompilerParams(
            use_tc_tiling_on_sc=False, needs_layout_passes=False),
        out_type=(jax.ShapeDtypeStruct((NC, NP, 16), jnp.float32),
                  jax.ShapeDtypeStruct((NC, NP, fw), jnp.float32)),
        scratch_types=[
            pltpu.VMEM((NSUB, SUB), jnp.int32),
            pltpu.VMEM((NSUB, SUB), jnp.int32),
            pltpu.VMEM((2, SUB, 16), jnp.float32),
            pltpu.VMEM((2, SUB, 16), jnp.float32),
            pltpu.VMEM((2, SUB, fw), jnp.float32),
            pltpu.VMEM((16,), jnp.float32),
            pltpu.VMEM_SHARED((NP, 16), jnp.float32),
            pltpu.VMEM_SHARED((NP, fw), jnp.float32),
            pltpu.SemaphoreType.DMA,
            pltpu.SemaphoreType.DMA,
        ])(functools.partial(_sc_layer, ngrp))
    return f(srcv, dstv, asd, g16, hf, z16, zf)


def _head_mat(a):
    heads, oc = a.shape
    eye = jnp.eye(heads, dtype=a.dtype)
    return (a[:, :, None] * eye[:, None, :]).reshape(heads * oc, heads)


def kernel(x, edge_index, edge_attr, batch, W1, a_src1, a_dst1, b1,
           Wm, a_srcm, a_dstm, bm, W2, a_src2, a_dst2, b2, lin1_W, lin2_W):
    del edge_attr, batch
    f32 = jnp.float32

    loops = jnp.arange(N, dtype=jnp.int32)
    src = jnp.concatenate([edge_index[0], loops])
    dst = jnp.concatenate([edge_index[1], loops])
    pad_e = EP - (E + N)
    pad_idx = N + jnp.arange(pad_e, dtype=jnp.int32) % (NP - N)
    srcv = jnp.concatenate([src, pad_idx]).reshape(NW, NSUB, SUB)
    dstv = jnp.concatenate([dst, pad_idx]).reshape(NW, NSUB, SUB)

    xp = jnp.pad(x, ((0, NP - N), (0, 2))).astype(f32)
    W1p = jnp.pad(W1, ((0, 2), (0, 0)))
    amat1 = jnp.concatenate([_head_mat(a_src1), _head_mat(a_dst1)], axis=1)
    amatm = jnp.concatenate([_head_mat(a_srcm), _head_mat(a_dstm)], axis=1)
    amat2 = jnp.concatenate([_head_mat(a_src2), _head_mat(a_dst2)], axis=1)
    lin2_W8 = jnp.pad(lin2_W, ((0, 0), (0, 7)))
    xmat128 = jnp.kron(jnp.eye(8, dtype=f32), jnp.ones((1, 16), f32))
    xmat64 = jnp.kron(jnp.eye(8, dtype=f32), jnp.ones((1, 8), f32))

    z16 = jnp.zeros((NP, 16), f32)
    z128 = jnp.zeros((NP, 128), f32)
    z64 = jnp.zeros((NP, 64), f32)

    h1, asd1, g1 = _tc_layer_pre(xp, W1p, amat1)
    s1, o1 = _sc_gat_layer(srcv, dstv, asd1, g1, h1, z16, z128)
    h2, asd2, g2 = _tc_layer_pre(
        None, Wm, amatm,
        fused_inputs=(o1, s1, xmat128, b1.reshape(1, 128)))
    s2, o2 = _sc_gat_layer(srcv, dstv, asd2, g2, h2, z16, z128)
    h3, asd3, g3 = _tc_layer_pre(
        None, W2, amat2,
        fused_inputs=(o2, s2, xmat128, bm.reshape(1, 128)))
    s3, o3 = _sc_gat_layer(srcv, dstv, asd3, g3, h3, z16, z64)
    out = _tc_head(o3, s3, xmat64, b2.reshape(1, 64), lin1_W, lin2_W8)
    return out[:N, :1]
